# baseline (speedup 1.0000x reference)
import functools

import jax
import jax.numpy as jnp
from jax import lax
from jax.experimental import pallas as pl
from jax.experimental.pallas import tpu as pltpu
from jax.experimental.pallas import tpu_sc as plsc

N = 10000
NP = 10240
E = 320000
EP = 327680
FEAT = 128
K = 16
OUT = 32
G = 64
NLVL = 3
NC = 2
NS = 16
NW = NC * NS
EW = EP // NW
CH = 1280
WPL = EP // CH
FAST = 0
NWF = 10
NWS = (WPL - NS * NWF) // NS
TOTWIN = NLVL * NWF
ACC_ROWS = NLVL * NP
ZROWS = ACC_ROWS // NS
CPL = NP // NS

_f32 = jnp.float32
_i32 = jnp.int32


def _t1_body(x_ref, th_ref, z_ref):
    x = x_ref[...]
    z_ref[...] = jnp.dot(jnp.maximum(x, 0.01 * x), th_ref[0],
                         preferred_element_type=_f32)


def _t1(x_all, th1_all):
    return pl.pallas_call(
        _t1_body,
        grid=(NLVL,),
        in_specs=[
            pl.BlockSpec((NP, FEAT), lambda i: (i, 0)),
            pl.BlockSpec((1, FEAT, K), lambda i: (i, 0, 0)),
        ],
        out_specs=pl.BlockSpec((NP, K), lambda i: (i, 0)),
        out_shape=jax.ShapeDtypeStruct((ACC_ROWS, K), _f32),
    )(x_all, th1_all)


def _spmm_body(mode, srcr, dstr, val_h, z_h, th_h, zc_h, out_h,
               acc, ztab, th_v, src_v, dst_v, val_v, rows_v, sems):
    cid = lax.axis_index("c")
    sid = lax.axis_index("s")
    zofs = sid * ZROWS
    is_fast = cid == FAST

    def ebase(t):
        l, j = divmod(t, NWF)
        wofs = jnp.where(is_fast, sid * NWF + j,
                         NS * NWF + sid * NWS + min(j, NWS - 1))
        return l * EP + wofs * CH

    def guard(t, fn):
        if t % NWF < NWS:
            fn()
        else:
            pl.when(is_fast)(fn)

    def issue_idx(t):
        fb = ebase(t)
        bi = t % 3
        return (
            pltpu.make_async_copy(srcr.at[pl.ds(fb, CH)], src_v.at[bi],
                                  sems.at[bi]),
            pltpu.make_async_copy(dstr.at[pl.ds(fb, CH)], dst_v.at[bi],
                                  sems.at[3 + bi]),
            pltpu.make_async_copy(val_h.at[pl.ds(fb, CH)], val_v.at[bi],
                                  sems.at[6 + bi]),
        )

    def start(cps):
        for cp in cps:
            cp.start()

    idx0 = issue_idx(0)
    start(idx0)
    zcp = pltpu.make_async_copy(zc_h.at[pl.ds(zofs, ZROWS)],
                                acc.at[pl.ds(zofs, ZROWS)], sems.at[11])
    zcp.start()
    if mode == 0:
        pltpu.sync_copy(z_h.at[pl.ds(zofs, ZROWS)],
                        ztab.at[pl.ds(zofs, ZROWS)])
    else:
        for l in range(NLVL):
            base = l * NP + sid * CPL
            pa = rows_v.at[0, pl.ds(0, CPL)]
            pb = rows_v.at[0, pl.ds(CPL, CPL)]
            zt = rows_v.at[1, pl.ds(0, CPL)]
            pltpu.sync_copy(z_h.at[0, pl.ds(base, CPL)], pa)
            pltpu.sync_copy(z_h.at[1, pl.ds(base, CPL)], pb)
            if mode == 1:
                pltpu.sync_copy(th_h.at[l], th_v)
                ths = [th_v[k] for k in range(K)]

                def mrow(r, carry):
                    a = rows_v[0, r] + rows_v[0, CPL + r]
                    a = jnp.maximum(a, 0.01 * a)
                    s = ths[0] * a[0]
                    for k in range(1, K):
                        s = s + ths[k] * a[k]
                    rows_v[1, r] = s
                    return carry
            else:

                def mrow(r, carry):
                    rows_v[1, r] = rows_v[0, r] + rows_v[0, CPL + r]
                    return carry

            lax.fori_loop(0, CPL, mrow, 0)
            pltpu.sync_copy(zt, ztab.at[pl.ds(base, CPL)])
    zcp.wait()
    plsc.subcore_barrier()

    idx_cps = {0: idx0}
    gat_cps = {}

    def wait_idx_start_gather(t):
        gat_cps[t] = pltpu.make_async_copy(
            ztab.at[src_v.at[t % 3]], rows_v.at[t % 2], sems.at[9 + t % 2])

        def fn():
            for cp in idx_cps[t]:
                cp.wait()
            gat_cps[t].start()

        guard(t, fn)

    wait_idx_start_gather(0)
    idx_cps[1] = issue_idx(1)
    guard(1, lambda: start(idx_cps[1]))

    for t in range(TOTWIN):
        bi = t % 3
        br = t % 2
        guard(t, lambda: gat_cps[t].wait())
        if t + 1 < TOTWIN:
            wait_idx_start_gather(t + 1)
        if t + 2 < TOTWIN:
            idx_cps[t + 2] = issue_idx(t + 2)
            guard(t + 2, lambda: start(idx_cps[t + 2]))

        def consume():
            def sbody(g, carry):
                e0 = g * 16
                v16 = val_v[bi, pl.ds(e0, 16)]
                for u in range(16):
                    rows_v[br, e0 + u] = rows_v[br, e0 + u] * v16[u]
                return carry

            lax.fori_loop(0, CH // 16, sbody, 0)
            pltpu.sync_copy(rows_v.at[br], acc.at[dst_v.at[bi]], add=True)

        guard(t, consume)

    plsc.subcore_barrier()
    pltpu.sync_copy(acc.at[pl.ds(zofs, ZROWS)],
                    out_h.at[cid, pl.ds(zofs, ZROWS)])


def _make_spmm(mode):
    return functools.partial(
        pl.kernel,
        out_type=jax.ShapeDtypeStruct((NC, ACC_ROWS, K), _f32),
        mesh=plsc.VectorSubcoreMesh(core_axis_name="c",
                                    subcore_axis_name="s"),
        compiler_params=pltpu.CompilerParams(use_tc_tiling_on_sc=False),
        scratch_types=[
            pltpu.VMEM_SHARED((ACC_ROWS, K), _f32),
            pltpu.VMEM_SHARED((ACC_ROWS, K), _f32),
            pltpu.VMEM((K, K), _f32),
            pltpu.VMEM((3, CH), _i32),
            pltpu.VMEM((3, CH), _i32),
            pltpu.VMEM((3, CH), _f32),
            pltpu.VMEM((2, CH, K), _f32),
            pltpu.SemaphoreType.DMA((12,)),
        ],
    )(functools.partial(_spmm_body, mode))


_spmm0 = _make_spmm(0)
_spmm1 = _make_spmm(1)
_spmm2 = _make_spmm(2)


def _t3_body(o_ref, batch_ref, th3_ref, w_ref, b_ref, out_ref):
    o = o_ref[0] + o_ref[1]
    acc = jnp.zeros((G, OUT), _f32)
    gi = lax.broadcasted_iota(_i32, (G, NP), 0)
    for l in range(NLVL):
        rows = o[l * NP:(l + 1) * NP]
        oh = (batch_ref[pl.ds(l, 1)] == gi).astype(_f32)
        psum = jnp.dot(oh, rows, preferred_element_type=_f32)
        cnt = jnp.sum(oh, axis=1, keepdims=True)
        p = jnp.dot(psum, th3_ref[l], preferred_element_type=_f32)
        acc = acc + p / jnp.maximum(cnt, 1.0)
    logits = lax.dot_general(acc, w_ref[...],
                             (((1,), (1,)), ((), ()))) + b_ref[...]
    m = jnp.max(logits, axis=1, keepdims=True)
    ex = jnp.exp(logits - m)
    out_ref[...] = ex / jnp.sum(ex, axis=1, keepdims=True)


def _t3(o_parts, batch_pad, th3_all, W, b2):
    return pl.pallas_call(
        _t3_body,
        out_shape=jax.ShapeDtypeStruct((G, OUT), _f32),
    )(o_parts, batch_pad, th3_all, W, b2)


def kernel(x0, x1, x2, l0_indices, l0_values, l1_indices, l1_values,
           l2_indices, l2_values, batch0, batch1, batch2,
           theta0_1, theta0_2, theta0_3, theta1_1, theta1_2, theta1_3,
           theta2_1, theta2_2, theta2_3, W, b):
    x_all = jnp.concatenate([
        jnp.pad(x, ((0, NP - N), (0, 0))) for x in (x0, x1, x2)
    ], axis=0)
    th1_all = jnp.stack([
        jnp.pad(t, ((0, 0), (0, K - t.shape[1])))
        for t in (theta0_1, theta1_1, theta2_1)
    ])
    th2_all = jnp.stack([
        jnp.pad(t, ((0, K - t.shape[0]), (0, K - t.shape[1])))
        for t in (theta0_2, theta1_2, theta2_2)
    ])
    th3_all = jnp.stack([
        jnp.pad(t, ((0, K - t.shape[0]), (0, 0)))
        for t in (theta0_3, theta1_3, theta2_3)
    ])

    def _pad_e(a, l):
        return jnp.pad(a, (0, EP - E)) + l * NP

    dst_adj = jnp.concatenate([
        _pad_e(idx[0], l)
        for l, idx in enumerate((l0_indices, l1_indices, l2_indices))
    ])
    src_adj = jnp.concatenate([
        _pad_e(idx[1], l)
        for l, idx in enumerate((l0_indices, l1_indices, l2_indices))
    ])
    val_all = jnp.concatenate([
        jnp.pad(v, (0, EP - E)) for v in (l0_values, l1_values, l2_values)
    ])

    batch_pad = jnp.stack([
        jnp.pad(bch, (0, NP - N), constant_values=G + 7)
        for bch in (batch0, batch1, batch2)
    ])
    b2 = b.reshape(1, OUT)
    zc = jnp.zeros((ACC_ROWS, K), _f32)
    thz = jnp.zeros((NLVL, K, K), _f32)

    z1 = _t1(x_all, th1_all)
    o1 = _spmm0(src_adj, dst_adj, val_all, z1, thz, zc)
    o2 = _spmm1(src_adj, dst_adj, val_all, o1, th2_all, zc)
    o3 = _spmm2(src_adj, dst_adj, val_all, o2, thz, zc)
    return _t3(o3, batch_pad, th3_all, W, b2)


# speedup vs baseline: 1.0043x; 1.0043x over previous
import functools

import jax
import jax.numpy as jnp
from jax import lax
from jax.experimental import pallas as pl
from jax.experimental.pallas import tpu as pltpu
from jax.experimental.pallas import tpu_sc as plsc

N = 10000
NP = 10240
E = 320000
EP = 327680
FEAT = 128
K = 16
OUT = 32
G = 64
NLVL = 3
NC = 2
NS = 16
NW = NC * NS
EW = EP // NW
CH = 1280
WPL = EP // CH
FAST = 1
NWF = 10
NWS = (WPL - NS * NWF) // NS
TOTWIN = NLVL * NWF
ACC_ROWS = NLVL * NP
ZROWS = ACC_ROWS // NS
CPL = NP // NS

_f32 = jnp.float32
_i32 = jnp.int32


def _t1_body(x_ref, th_ref, z_ref):
    x = x_ref[...]
    z_ref[...] = jnp.dot(jnp.maximum(x, 0.01 * x), th_ref[0],
                         preferred_element_type=_f32)


def _t1(x_all, th1_all):
    return pl.pallas_call(
        _t1_body,
        grid=(NLVL,),
        in_specs=[
            pl.BlockSpec((NP, FEAT), lambda i: (i, 0)),
            pl.BlockSpec((1, FEAT, K), lambda i: (i, 0, 0)),
        ],
        out_specs=pl.BlockSpec((NP, K), lambda i: (i, 0)),
        out_shape=jax.ShapeDtypeStruct((ACC_ROWS, K), _f32),
    )(x_all, th1_all)


def _spmm_body(mode, srcr, dstr, val_h, z_h, th_h, zc_h, out_h,
               acc, ztab, th_v, src_v, dst_v, val_v, rows_v, sems):
    cid = lax.axis_index("c")
    sid = lax.axis_index("s")
    zofs = sid * ZROWS
    is_fast = cid == FAST

    def ebase(t):
        l, j = divmod(t, NWF)
        wofs = jnp.where(is_fast, sid * NWF + j,
                         NS * NWF + sid * NWS + min(j, NWS - 1))
        return l * EP + wofs * CH

    def guard(t, fn):
        if t % NWF < NWS:
            fn()
        else:
            pl.when(is_fast)(fn)

    def issue_idx(t):
        fb = ebase(t)
        bi = t % 3
        return (
            pltpu.make_async_copy(srcr.at[pl.ds(fb, CH)], src_v.at[bi],
                                  sems.at[bi]),
            pltpu.make_async_copy(dstr.at[pl.ds(fb, CH)], dst_v.at[bi],
                                  sems.at[3 + bi]),
            pltpu.make_async_copy(val_h.at[pl.ds(fb, CH)], val_v.at[bi],
                                  sems.at[6 + bi]),
        )

    def start(cps):
        for cp in cps:
            cp.start()

    idx0 = issue_idx(0)
    start(idx0)
    zcp = pltpu.make_async_copy(zc_h.at[pl.ds(zofs, ZROWS)],
                                acc.at[pl.ds(zofs, ZROWS)], sems.at[11])
    zcp.start()
    if mode == 0:
        pltpu.sync_copy(z_h.at[pl.ds(zofs, ZROWS)],
                        ztab.at[pl.ds(zofs, ZROWS)])
    else:
        for l in range(NLVL):
            base = l * NP + sid * CPL
            pa = rows_v.at[0, pl.ds(0, CPL)]
            pb = rows_v.at[0, pl.ds(CPL, CPL)]
            zt = rows_v.at[1, pl.ds(0, CPL)]
            pltpu.sync_copy(z_h.at[0, pl.ds(base, CPL)], pa)
            pltpu.sync_copy(z_h.at[1, pl.ds(base, CPL)], pb)
            if mode == 1:
                pltpu.sync_copy(th_h.at[l], th_v)
                ths = [th_v[k] for k in range(K)]

                def mrow(r, carry):
                    a = rows_v[0, r] + rows_v[0, CPL + r]
                    a = jnp.maximum(a, 0.01 * a)
                    s = ths[0] * a[0]
                    for k in range(1, K):
                        s = s + ths[k] * a[k]
                    rows_v[1, r] = s
                    return carry
            else:

                def mrow(r, carry):
                    rows_v[1, r] = rows_v[0, r] + rows_v[0, CPL + r]
                    return carry

            lax.fori_loop(0, CPL, mrow, 0)
            pltpu.sync_copy(zt, ztab.at[pl.ds(base, CPL)])
    zcp.wait()
    plsc.subcore_barrier()

    idx_cps = {0: idx0}
    gat_cps = {}

    def wait_idx_start_gather(t):
        gat_cps[t] = pltpu.make_async_copy(
            ztab.at[src_v.at[t % 3]], rows_v.at[t % 2], sems.at[9 + t % 2])

        def fn():
            for cp in idx_cps[t]:
                cp.wait()
            gat_cps[t].start()

        guard(t, fn)

    wait_idx_start_gather(0)
    idx_cps[1] = issue_idx(1)
    guard(1, lambda: start(idx_cps[1]))

    for t in range(TOTWIN):
        bi = t % 3
        br = t % 2
        guard(t, lambda: gat_cps[t].wait())
        if t + 1 < TOTWIN:
            wait_idx_start_gather(t + 1)
        if t + 2 < TOTWIN:
            idx_cps[t + 2] = issue_idx(t + 2)
            guard(t + 2, lambda: start(idx_cps[t + 2]))

        def consume():
            def sbody(g, carry):
                e0 = g * 16
                v16 = val_v[bi, pl.ds(e0, 16)]
                for u in range(16):
                    rows_v[br, e0 + u] = rows_v[br, e0 + u] * v16[u]
                return carry

            lax.fori_loop(0, CH // 16, sbody, 0)
            pltpu.sync_copy(rows_v.at[br], acc.at[dst_v.at[bi]], add=True)

        guard(t, consume)

    plsc.subcore_barrier()
    pltpu.sync_copy(acc.at[pl.ds(zofs, ZROWS)],
                    out_h.at[cid, pl.ds(zofs, ZROWS)])


def _make_spmm(mode):
    return functools.partial(
        pl.kernel,
        out_type=jax.ShapeDtypeStruct((NC, ACC_ROWS, K), _f32),
        mesh=plsc.VectorSubcoreMesh(core_axis_name="c",
                                    subcore_axis_name="s"),
        compiler_params=pltpu.CompilerParams(use_tc_tiling_on_sc=False),
        scratch_types=[
            pltpu.VMEM_SHARED((ACC_ROWS, K), _f32),
            pltpu.VMEM_SHARED((ACC_ROWS, K), _f32),
            pltpu.VMEM((K, K), _f32),
            pltpu.VMEM((3, CH), _i32),
            pltpu.VMEM((3, CH), _i32),
            pltpu.VMEM((3, CH), _f32),
            pltpu.VMEM((2, CH, K), _f32),
            pltpu.SemaphoreType.DMA((12,)),
        ],
    )(functools.partial(_spmm_body, mode))


_spmm0 = _make_spmm(0)
_spmm1 = _make_spmm(1)
_spmm2 = _make_spmm(2)


def _t3_body(o_ref, batch_ref, th3_ref, w_ref, b_ref, out_ref):
    o = o_ref[0] + o_ref[1]
    acc = jnp.zeros((G, OUT), _f32)
    gi = lax.broadcasted_iota(_i32, (G, NP), 0)
    for l in range(NLVL):
        rows = o[l * NP:(l + 1) * NP]
        oh = (batch_ref[pl.ds(l, 1)] == gi).astype(_f32)
        psum = jnp.dot(oh, rows, preferred_element_type=_f32)
        cnt = jnp.sum(oh, axis=1, keepdims=True)
        p = jnp.dot(psum, th3_ref[l], preferred_element_type=_f32)
        acc = acc + p / jnp.maximum(cnt, 1.0)
    logits = lax.dot_general(acc, w_ref[...],
                             (((1,), (1,)), ((), ()))) + b_ref[...]
    m = jnp.max(logits, axis=1, keepdims=True)
    ex = jnp.exp(logits - m)
    out_ref[...] = ex / jnp.sum(ex, axis=1, keepdims=True)


def _t3(o_parts, batch_pad, th3_all, W, b2):
    return pl.pallas_call(
        _t3_body,
        out_shape=jax.ShapeDtypeStruct((G, OUT), _f32),
    )(o_parts, batch_pad, th3_all, W, b2)


def kernel(x0, x1, x2, l0_indices, l0_values, l1_indices, l1_values,
           l2_indices, l2_values, batch0, batch1, batch2,
           theta0_1, theta0_2, theta0_3, theta1_1, theta1_2, theta1_3,
           theta2_1, theta2_2, theta2_3, W, b):
    x_all = jnp.concatenate([
        jnp.pad(x, ((0, NP - N), (0, 0))) for x in (x0, x1, x2)
    ], axis=0)
    th1_all = jnp.stack([
        jnp.pad(t, ((0, 0), (0, K - t.shape[1])))
        for t in (theta0_1, theta1_1, theta2_1)
    ])
    th2_all = jnp.stack([
        jnp.pad(t, ((0, K - t.shape[0]), (0, K - t.shape[1])))
        for t in (theta0_2, theta1_2, theta2_2)
    ])
    th3_all = jnp.stack([
        jnp.pad(t, ((0, K - t.shape[0]), (0, 0)))
        for t in (theta0_3, theta1_3, theta2_3)
    ])

    def _pad_e(a, l):
        return jnp.pad(a, (0, EP - E)) + l * NP

    dst_adj = jnp.concatenate([
        _pad_e(idx[0], l)
        for l, idx in enumerate((l0_indices, l1_indices, l2_indices))
    ])
    src_adj = jnp.concatenate([
        _pad_e(idx[1], l)
        for l, idx in enumerate((l0_indices, l1_indices, l2_indices))
    ])
    val_all = jnp.concatenate([
        jnp.pad(v, (0, EP - E)) for v in (l0_values, l1_values, l2_values)
    ])

    batch_pad = jnp.stack([
        jnp.pad(bch, (0, NP - N), constant_values=G + 7)
        for bch in (batch0, batch1, batch2)
    ])
    b2 = b.reshape(1, OUT)
    zc = jnp.zeros((ACC_ROWS, K), _f32)
    thz = jnp.zeros((NLVL, K, K), _f32)

    z1 = _t1(x_all, th1_all)
    o1 = _spmm0(src_adj, dst_adj, val_all, z1, thz, zc)
    o2 = _spmm1(src_adj, dst_adj, val_all, o1, th2_all, zc)
    o3 = _spmm2(src_adj, dst_adj, val_all, o2, thz, zc)
    return _t3(o3, batch_pad, th3_all, W, b2)


# speedup vs baseline: 1.0218x; 1.0175x over previous
import functools

import jax
import jax.numpy as jnp
from jax import lax
from jax.experimental import pallas as pl
from jax.experimental.pallas import tpu as pltpu
from jax.experimental.pallas import tpu_sc as plsc

N = 10000
NP = 10240
E = 320000
EP = 327680
FEAT = 128
K = 16
OUT = 32
G = 64
NLVL = 3
NC = 2
NS = 16
NW = NC * NS
EW = EP // NW
CH = 1280
WPL = EP // CH
FAST = 1
NWF = 10
NWS = (WPL - NS * NWF) // NS
TOTWIN = NLVL * NWF
ACC_ROWS = NLVL * NP
ZROWS = ACC_ROWS // NS
CPL = NP // NS
QSTR = 72
Q_ROWS = 224

_f32 = jnp.float32
_i32 = jnp.int32


def _t1_body(x_ref, th_ref, z_ref):
    x = x_ref[...]
    z_ref[...] = jnp.dot(jnp.maximum(x, 0.01 * x), th_ref[0],
                         preferred_element_type=_f32)


def _t1(x_all, th1_all):
    return pl.pallas_call(
        _t1_body,
        grid=(NLVL,),
        in_specs=[
            pl.BlockSpec((NP, FEAT), lambda i: (i, 0)),
            pl.BlockSpec((1, FEAT, K), lambda i: (i, 0, 0)),
        ],
        out_specs=pl.BlockSpec((NP, K), lambda i: (i, 0)),
        out_shape=jax.ShapeDtypeStruct((ACC_ROWS, K), _f32),
    )(x_all, th1_all)


def _spmm_body(mode, srcr, dstr, val_h, z_h, th_h, zc_h, batch_h, out_h,
               acc, ztab, th_v, src_v, dst_v, val_v, rows_v, bat_v, q_v,
               sems):
    cid = lax.axis_index("c")
    sid = lax.axis_index("s")
    zofs = sid * ZROWS
    is_fast = cid == FAST

    def ebase(t):
        l, j = divmod(t, NWF)
        wofs = jnp.where(is_fast, sid * NWF + j,
                         NS * NWF + sid * NWS + min(j, NWS - 1))
        return l * EP + wofs * CH

    def guard(t, fn):
        if t % NWF < NWS:
            fn()
        else:
            pl.when(is_fast)(fn)

    def issue_idx(t):
        fb = ebase(t)
        bi = t % 3
        return (
            pltpu.make_async_copy(srcr.at[pl.ds(fb, CH)], src_v.at[bi],
                                  sems.at[bi]),
            pltpu.make_async_copy(dstr.at[pl.ds(fb, CH)], dst_v.at[bi],
                                  sems.at[3 + bi]),
            pltpu.make_async_copy(val_h.at[pl.ds(fb, CH)], val_v.at[bi],
                                  sems.at[6 + bi]),
        )

    def start(cps):
        for cp in cps:
            cp.start()

    idx0 = issue_idx(0)
    start(idx0)
    zcp = pltpu.make_async_copy(zc_h.at[pl.ds(zofs, ZROWS)],
                                acc.at[pl.ds(zofs, ZROWS)], sems.at[11])
    zcp.start()
    if mode == 0:
        pltpu.sync_copy(z_h.at[pl.ds(zofs, ZROWS)],
                        ztab.at[pl.ds(zofs, ZROWS)])
    else:
        for l in range(NLVL):
            base = l * NP + sid * CPL
            pa = rows_v.at[0, pl.ds(0, CPL)]
            pb = rows_v.at[0, pl.ds(CPL, CPL)]
            zt = rows_v.at[1, pl.ds(0, CPL)]
            pltpu.sync_copy(z_h.at[0, pl.ds(base, CPL)], pa)
            pltpu.sync_copy(z_h.at[1, pl.ds(base, CPL)], pb)
            if mode == 1:
                pltpu.sync_copy(th_h.at[l], th_v)
                ths = [th_v[k] for k in range(K)]

                def mrow(r, carry):
                    a = rows_v[0, r] + rows_v[0, CPL + r]
                    a = jnp.maximum(a, 0.01 * a)
                    s = ths[0] * a[0]
                    for k in range(1, K):
                        s = s + ths[k] * a[k]
                    rows_v[1, r] = s
                    return carry
            else:

                def mrow(r, carry):
                    rows_v[1, r] = rows_v[0, r] + rows_v[0, CPL + r]
                    return carry

            lax.fori_loop(0, CPL, mrow, 0)
            pltpu.sync_copy(zt, ztab.at[pl.ds(base, CPL)])
    zcp.wait()
    plsc.subcore_barrier()

    idx_cps = {0: idx0}
    gat_cps = {}

    def wait_idx_start_gather(t):
        gat_cps[t] = pltpu.make_async_copy(
            ztab.at[src_v.at[t % 3]], rows_v.at[t % 2], sems.at[9 + t % 2])

        def fn():
            for cp in idx_cps[t]:
                cp.wait()
            gat_cps[t].start()

        guard(t, fn)

    wait_idx_start_gather(0)
    idx_cps[1] = issue_idx(1)
    guard(1, lambda: start(idx_cps[1]))

    for t in range(TOTWIN):
        bi = t % 3
        br = t % 2
        guard(t, lambda: gat_cps[t].wait())
        if t + 1 < TOTWIN:
            wait_idx_start_gather(t + 1)
        if t + 2 < TOTWIN:
            idx_cps[t + 2] = issue_idx(t + 2)
            guard(t + 2, lambda: start(idx_cps[t + 2]))

        def consume():
            def sbody(g, carry):
                e0 = g * 16
                v16 = val_v[bi, pl.ds(e0, 16)]
                for u in range(16):
                    rows_v[br, e0 + u] = rows_v[br, e0 + u] * v16[u]
                return carry

            lax.fori_loop(0, CH // 16, sbody, 0)
            pltpu.sync_copy(rows_v.at[br], acc.at[dst_v.at[bi]], add=True)

        guard(t, consume)

    plsc.subcore_barrier()
    if mode != 2:
        pltpu.sync_copy(acc.at[pl.ds(zofs, ZROWS)],
                        out_h.at[cid, pl.ds(zofs, ZROWS)])
    else:
        lane = lax.iota(_i32, 16)
        e15 = jnp.where(lane == 15, 1.0, 0.0).astype(_f32)
        zero16 = jnp.zeros((16,), _f32)

        def zrow(r, carry):
            q_v[r] = zero16
            return carry

        lax.fori_loop(0, Q_ROWS, zrow, 0)
        for l in range(NLVL):
            base = l * NP + sid * CPL
            pltpu.sync_copy(acc.at[pl.ds(base, CPL)],
                            rows_v.at[0, pl.ds(0, CPL)])
            pltpu.sync_copy(batch_h.at[l, pl.ds(sid * CPL, CPL)], bat_v)

            def prow(i, carry):
                r0 = i * 16
                b16 = bat_v[pl.ds(r0, 16)]
                for u in range(16):
                    g = b16[u] + l * QSTR
                    q_v[g] = q_v[g] + (rows_v[0, r0 + u] + e15)
                return carry

            lax.fori_loop(0, CPL // 16, prow, 0)
        pltpu.sync_copy(q_v, out_h.at[cid, sid])


def _make_spmm(mode):
    out_t = (jax.ShapeDtypeStruct((NC, NS, Q_ROWS, K), _f32) if mode == 2
             else jax.ShapeDtypeStruct((NC, ACC_ROWS, K), _f32))
    return functools.partial(
        pl.kernel,
        out_type=out_t,
        mesh=plsc.VectorSubcoreMesh(core_axis_name="c",
                                    subcore_axis_name="s"),
        compiler_params=pltpu.CompilerParams(use_tc_tiling_on_sc=False),
        scratch_types=[
            pltpu.VMEM_SHARED((ACC_ROWS, K), _f32),
            pltpu.VMEM_SHARED((ACC_ROWS, K), _f32),
            pltpu.VMEM((K, K), _f32),
            pltpu.VMEM((3, CH), _i32),
            pltpu.VMEM((3, CH), _i32),
            pltpu.VMEM((3, CH), _f32),
            pltpu.VMEM((2, CH, K), _f32),
            pltpu.VMEM((CPL,), _i32),
            pltpu.VMEM((Q_ROWS, K), _f32),
            pltpu.SemaphoreType.DMA((12,)),
        ],
    )(functools.partial(_spmm_body, mode))


_spmm0 = _make_spmm(0)
_spmm1 = _make_spmm(1)
_spmm2 = _make_spmm(2)


def _t3_body(q_ref, th3_ref, w_ref, b_ref, out_ref):
    q = q_ref[0, 0]
    for c in range(NC):
        for s in range(NS):
            if c or s:
                q = q + q_ref[c, s]
    sel = jnp.where(lax.broadcasted_iota(_i32, (K, 1), 0) == 15, 1.0,
                    0.0).astype(_f32)
    acc = jnp.zeros((G, OUT), _f32)
    for l in range(NLVL):
        ql = q[l * QSTR:l * QSTR + G]
        cnt = jnp.dot(ql, sel, preferred_element_type=_f32)
        p = jnp.dot(ql, th3_ref[l], preferred_element_type=_f32)
        acc = acc + p / jnp.maximum(cnt, 1.0)
    logits = lax.dot_general(acc, w_ref[...],
                             (((1,), (1,)), ((), ()))) + b_ref[...]
    m = jnp.max(logits, axis=1, keepdims=True)
    ex = jnp.exp(logits - m)
    out_ref[...] = ex / jnp.sum(ex, axis=1, keepdims=True)


def _t3(q_parts, th3_all, W, b2):
    return pl.pallas_call(
        _t3_body,
        out_shape=jax.ShapeDtypeStruct((G, OUT), _f32),
    )(q_parts, th3_all, W, b2)


def kernel(x0, x1, x2, l0_indices, l0_values, l1_indices, l1_values,
           l2_indices, l2_values, batch0, batch1, batch2,
           theta0_1, theta0_2, theta0_3, theta1_1, theta1_2, theta1_3,
           theta2_1, theta2_2, theta2_3, W, b):
    x_all = jnp.concatenate([
        jnp.pad(x, ((0, NP - N), (0, 0))) for x in (x0, x1, x2)
    ], axis=0)
    th1_all = jnp.stack([
        jnp.pad(t, ((0, 0), (0, K - t.shape[1])))
        for t in (theta0_1, theta1_1, theta2_1)
    ])
    th2_all = jnp.stack([
        jnp.pad(t, ((0, K - t.shape[0]), (0, K - t.shape[1])))
        for t in (theta0_2, theta1_2, theta2_2)
    ])
    th3_all = jnp.stack([
        jnp.pad(t, ((0, K - t.shape[0]), (0, 0)))
        for t in (theta0_3, theta1_3, theta2_3)
    ])

    def _pad_e(a, l):
        return jnp.pad(a, (0, EP - E)) + l * NP

    dst_adj = jnp.concatenate([
        _pad_e(idx[0], l)
        for l, idx in enumerate((l0_indices, l1_indices, l2_indices))
    ])
    src_adj = jnp.concatenate([
        _pad_e(idx[1], l)
        for l, idx in enumerate((l0_indices, l1_indices, l2_indices))
    ])
    val_all = jnp.concatenate([
        jnp.pad(v, (0, EP - E)) for v in (l0_values, l1_values, l2_values)
    ])

    batch_pad = jnp.stack([
        jnp.pad(bch, (0, NP - N), constant_values=G)
        for bch in (batch0, batch1, batch2)
    ])
    b2 = b.reshape(1, OUT)
    zc = jnp.zeros((ACC_ROWS, K), _f32)
    thz = jnp.zeros((NLVL, K, K), _f32)

    z1 = _t1(x_all, th1_all)
    o1 = _spmm0(src_adj, dst_adj, val_all, z1, thz, zc, batch_pad)
    o2 = _spmm1(src_adj, dst_adj, val_all, o1, th2_all, zc, batch_pad)
    q = _spmm2(src_adj, dst_adj, val_all, o2, thz, zc, batch_pad)
    return _t3(q, th3_all, W, b2)
